# Initial kernel scaffold; baseline (speedup 1.0000x reference)
import jax
import jax.numpy as jnp
from jax.experimental import pallas as pl


def kernel(x, table):
    raise NotImplementedError("write your pallas kernel here")


# speedup vs baseline: 2.9651x; 2.9651x over previous
import functools

import jax
import jax.numpy as jnp
from jax import lax
from jax.experimental import pallas as pl
from jax.experimental.pallas import tpu as pltpu
from jax.experimental.pallas import tpu_sc as plsc

D_MODEL = 128
NC, NS = 2, 16
NW = NC * NS
G = 128


@functools.lru_cache(maxsize=None)
def _make_gather(nb: int, vocab: int):
    bpw = nb // NW
    ng = bpw // G

    mesh = plsc.VectorSubcoreMesh(core_axis_name="c", subcore_axis_name="s")

    @functools.partial(
        pl.kernel,
        mesh=mesh,
        out_type=jax.ShapeDtypeStruct((nb, D_MODEL), jnp.float32),
        scratch_types=[
            pltpu.VMEM((ng, G), jnp.int32),
            pltpu.VMEM((G, D_MODEL), jnp.float32),
            pltpu.SemaphoreType.DMA,
        ],
    )
    def gather(table_hbm, idx_hbm, out_hbm, idx_v, rows_v, sem):
        wid = lax.axis_index("s") * NC + lax.axis_index("c")
        pltpu.sync_copy(idx_hbm.at[wid], idx_v)
        base = wid * bpw

        def body(g, carry):
            pltpu.async_copy(table_hbm.at[idx_v.at[g]], rows_v, sem).wait()
            pltpu.sync_copy(rows_v, out_hbm.at[pl.ds(base + g * G, G)])
            return carry

        lax.fori_loop(0, ng, body, 0)

    return gather


def kernel(x, table):
    b, h = x.shape
    nb = b * h
    idx3 = x.reshape(NW, nb // NW // G, G)
    out = _make_gather(nb, table.shape[0])(table, idx3)
    return out.reshape(b, h, D_MODEL)


# speedup vs baseline: 3.3314x; 1.1236x over previous
import functools

import jax
import jax.numpy as jnp
from jax import lax
from jax.experimental import pallas as pl
from jax.experimental.pallas import tpu as pltpu
from jax.experimental.pallas import tpu_sc as plsc

D_MODEL = 128
NC, NS = 2, 16
NW = NC * NS
G = 80
K = 4


@functools.lru_cache(maxsize=None)
def _make_gather(nb: int, vocab: int):
    bpw = nb // NW
    ng = bpw // G
    nwin = ng // K
    npair = nwin // 2
    assert ng % K == 0 and nwin % 2 == 0

    mesh = plsc.VectorSubcoreMesh(core_axis_name="c", subcore_axis_name="s")

    @functools.partial(
        pl.kernel,
        mesh=mesh,
        out_type=jax.ShapeDtypeStruct((nb, D_MODEL), jnp.float32),
        scratch_types=[
            pltpu.VMEM((ng, G), jnp.int32),
            pltpu.VMEM((2, K, G, D_MODEL), jnp.float32),
            pltpu.SemaphoreType.DMA,
            pltpu.SemaphoreType.DMA,
            pltpu.SemaphoreType.DMA,
        ],
    )
    def gather(table_hbm, idx_hbm, out_hbm, idx_v, rows_v, gsem0, gsem1, ssem):
        wid = lax.axis_index("s") * NC + lax.axis_index("c")
        pltpu.sync_copy(idx_hbm.at[wid], idx_v)
        base = wid * bpw
        gsems = (gsem0, gsem1)

        def start_gather(g, s, b):
            pltpu.async_copy(table_hbm.at[idx_v.at[g]], rows_v.at[s, b], gsems[s])

        def start_store(g, s, b):
            pltpu.async_copy(rows_v.at[s, b], out_hbm.at[pl.ds(base + g * G, G)], ssem)

        def wait_one(sem, s, b):
            pltpu.make_async_copy(rows_v.at[s, b], out_hbm.at[pl.ds(base, G)], sem).wait()

        def window(w, s, first, last):
            def drain_prev():
                for b in range(K):
                    wait_one(ssem, 1 - s, b)
            if first is None:
                drain_prev()
            else:
                pl.when(jnp.logical_not(first))(drain_prev)

            def next_gathers():
                for b in range(K):
                    start_gather((w + 1) * K + b, 1 - s, b)
            if last is None:
                next_gathers()
            else:
                pl.when(jnp.logical_not(last))(next_gathers)

            for b in range(K):
                wait_one(gsems[s], s, b)
            for b in range(K):
                start_store(w * K + b, s, b)

        for b in range(K):
            start_gather(b, 0, b)

        def pair(p, carry):
            window(2 * p, 0, first=(p == 0), last=None)
            window(2 * p + 1, 1, first=None, last=(p == npair - 1))
            return carry

        lax.fori_loop(0, npair, pair, 0)

        for b in range(K):
            wait_one(ssem, 1, b)

    return gather


def kernel(x, table):
    b, h = x.shape
    nb = b * h
    idx3 = x.reshape(NW, nb // NW // G, G)
    out = _make_gather(nb, table.shape[0])(table, idx3)
    return out.reshape(b, h, D_MODEL)


# speedup vs baseline: 5.9373x; 1.7822x over previous
import functools

import jax
import jax.numpy as jnp
from jax import lax
from jax.experimental import pallas as pl
from jax.experimental.pallas import tpu as pltpu
from jax.experimental.pallas import tpu_sc as plsc

NC, NS = 2, 16
NW = NC * NS
K = 8


@functools.lru_cache(maxsize=None)
def _make_gather(batch: int, hist: int, d_model: int):
    spw = batch // NW
    nwin = spw // K
    npair = nwin // 2
    assert spw % K == 0 and nwin % 2 == 0

    mesh = plsc.VectorSubcoreMesh(core_axis_name="c", subcore_axis_name="s")

    @functools.partial(
        pl.kernel,
        mesh=mesh,
        out_type=jax.ShapeDtypeStruct((batch, hist, d_model), jnp.float32),
        scratch_types=[
            pltpu.VMEM((spw, hist), jnp.int32),
            pltpu.VMEM((2, K, hist, d_model), jnp.float32),
            pltpu.SemaphoreType.DMA,
            pltpu.SemaphoreType.DMA,
            pltpu.SemaphoreType.DMA,
        ],
    )
    def gather(table_hbm, idx_hbm, out_hbm, idx_v, rows_v, gsem0, gsem1, ssem):
        wid = lax.axis_index("s") * NC + lax.axis_index("c")
        base = wid * spw
        pltpu.sync_copy(idx_hbm.at[pl.ds(base, spw)], idx_v)
        gsems = (gsem0, gsem1)

        def start_gather(g, s, b):
            pltpu.async_copy(table_hbm.at[idx_v.at[g]], rows_v.at[s, b], gsems[s])

        def start_store(g, s, b):
            pltpu.async_copy(rows_v.at[s, b], out_hbm.at[base + g], ssem)

        def wait_one(sem, s, b):
            pltpu.make_async_copy(rows_v.at[s, b], out_hbm.at[base], sem).wait()

        def window(w, s, first, last):
            def drain_prev():
                for b in range(K):
                    wait_one(ssem, 1 - s, b)
            if first is None:
                drain_prev()
            else:
                pl.when(jnp.logical_not(first))(drain_prev)

            def next_gathers():
                for b in range(K):
                    start_gather((w + 1) * K + b, 1 - s, b)
            if last is None:
                next_gathers()
            else:
                pl.when(jnp.logical_not(last))(next_gathers)

            for b in range(K):
                wait_one(gsems[s], s, b)
            for b in range(K):
                start_store(w * K + b, s, b)

        for b in range(K):
            start_gather(b, 0, b)

        def pair(p, carry):
            window(2 * p, 0, first=(p == 0), last=None)
            window(2 * p + 1, 1, first=None, last=(p == npair - 1))
            return carry

        lax.fori_loop(0, npair, pair, 0)

        for b in range(K):
            wait_one(ssem, 1, b)

    return gather


def kernel(x, table):
    b, h = x.shape
    return _make_gather(b, h, table.shape[1])(table, x)
